# baseline (speedup 1.0000x reference)
import functools

import jax
import jax.numpy as jnp
from jax import lax
from jax.experimental import pallas as pl
from jax.experimental.pallas import tpu as pltpu
from jax.experimental.pallas import tpu_sc as plsc

T = 16384
K = 7
KP = 8
NE = 64
INV_ROUTED_SCALING = 1.0 / 1.5

try:
    _INFO = plsc.get_sparse_core_info()
    _NC = _INFO.num_cores
    _NS = _INFO.num_subcores
except Exception:
    _NC, _NS = 2, 16
_NW = _NC * _NS
_CHUNK = 128
_ROWS = T // _CHUNK
_RPW = _ROWS // _NW


@functools.cache
def _build_gather_ids():
    @functools.partial(
        pl.kernel,
        mesh=plsc.VectorSubcoreMesh(core_axis_name="c", subcore_axis_name="s"),
        out_type=jax.ShapeDtypeStruct((_ROWS, _CHUNK, KP), jnp.int32),
        scratch_types=[
            pltpu.VMEM((_RPW, _CHUNK), jnp.int32),
            pltpu.VMEM((_RPW, _CHUNK, KP), jnp.int32),
            pltpu.SemaphoreType.DMA,
        ],
        compiler_params=pltpu.CompilerParams(use_tc_tiling_on_sc=False),
    )
    def _gather_ids(idx_hbm, table_hbm, out_hbm, idx_v, rows_v, sem):
        wid = lax.axis_index("s") * _NC + lax.axis_index("c")
        base = wid * _RPW
        pltpu.sync_copy(idx_hbm.at[pl.ds(base, _RPW)], idx_v)
        copies = [
            pltpu.async_copy(table_hbm.at[idx_v.at[j]], rows_v.at[j], sem)
            for j in range(_RPW)
        ]
        for c in copies:
            c.wait()
        pltpu.sync_copy(rows_v, out_hbm.at[pl.ds(base, _RPW)])

    return _gather_ids


_BLK = 2048


def _combine_body(logits_ref, ids_ref, w_ref, id_ref):
    x = logits_ref[...]
    s = jnp.sqrt(jax.nn.softplus(x))
    ids = ids_ref[...]
    col = lax.broadcasted_iota(jnp.int32, (_BLK, NE), 1)
    ws = []
    for j in range(K):
        idj = lax.slice(ids, (0, j), (_BLK, j + 1))
        wj = jnp.sum(jnp.where(col == idj, s, 0.0), axis=1, keepdims=True)
        ws.append(wj)
    wsum = ws[0]
    for wj in ws[1:]:
        wsum = wsum + wj
    inv = 1.0 / wsum
    wn = [wj * inv for wj in ws]
    sn = wn[0]
    for wj in wn[1:]:
        sn = sn + wj
    shared = sn * INV_ROUTED_SCALING
    w_ref[...] = jnp.concatenate(wn + [shared], axis=1)
    id_ref[...] = jnp.concatenate(
        [lax.slice(ids, (0, 0), (_BLK, K)),
         jnp.full((_BLK, 1), NE, jnp.int32)], axis=1)


_combine = pl.pallas_call(
    _combine_body,
    grid=(T // _BLK,),
    in_specs=[
        pl.BlockSpec((_BLK, NE), lambda i: (i, 0)),
        pl.BlockSpec((_BLK, KP), lambda i: (i, 0)),
    ],
    out_specs=[
        pl.BlockSpec((_BLK, K + 1), lambda i: (i, 0)),
        pl.BlockSpec((_BLK, K + 1), lambda i: (i, 0)),
    ],
    out_shape=[
        jax.ShapeDtypeStruct((T, K + 1), jnp.float32),
        jax.ShapeDtypeStruct((T, K + 1), jnp.int32),
    ],
)


def kernel(hidden_states, router_logits, input_ids, tid2eid):
    del hidden_states
    table = jnp.pad(tid2eid, ((0, 0), (0, KP - K)))
    ids_main = _build_gather_ids()(input_ids.reshape(_ROWS, _CHUNK), table)
    ids_main = ids_main.reshape(T, KP)
    topk_weights, topk_ids = _combine(router_logits, ids_main)
    return topk_weights, topk_ids, router_logits


# speedup vs baseline: 1.4245x; 1.4245x over previous
import functools

import jax
import jax.numpy as jnp
from jax import lax
from jax.experimental import pallas as pl
from jax.experimental.pallas import tpu as pltpu
from jax.experimental.pallas import tpu_sc as plsc

T = 16384
K = 7
NE = 64
INV_ROUTED_SCALING = 1.0 / 1.5
TAB_ROWS = 100000 * K // 8
TPW = 512

_P = (0.9999819, -0.49918786, 0.3244118, -0.20866966, 0.10028721,
      -0.023689253)

try:
    _INFO = plsc.get_sparse_core_info()
    _NC = _INFO.num_cores
    _NS = _INFO.num_subcores
except Exception:
    _NC, _NS = 2, 16


def _sqrt_softplus(x):
    e = jnp.exp(-jnp.abs(x))
    acc = jnp.full((16,), _P[5], jnp.float32)
    for k in range(4, -1, -1):
        acc = acc * e + _P[k]
    sp = jnp.maximum(x, 0.0) + acc * e
    b = plsc.bitcast(sp, jnp.int32)
    q = plsc.bitcast(0x5F3759DF - lax.shift_right_logical(b, 1), jnp.float32)
    for _ in range(3):
        q = q * (1.5 - 0.5 * sp * q * q)
    return sp * q


@functools.cache
def _build_route():
    @functools.partial(
        pl.kernel,
        mesh=plsc.VectorSubcoreMesh(core_axis_name="c", subcore_axis_name="s"),
        out_type=[
            jax.ShapeDtypeStruct((T, K + 1), jnp.float32),
            jax.ShapeDtypeStruct((T, K + 1), jnp.int32),
        ],
        scratch_types=[
            pltpu.VMEM((TPW,), jnp.int32),
            pltpu.VMEM((TPW,), jnp.int32),
            pltpu.VMEM((8, 128), jnp.int32),
            pltpu.VMEM((8, 128, 8), jnp.int32),
            pltpu.VMEM((TPW * NE,), jnp.float32),
            pltpu.VMEM((TPW, K + 1), jnp.float32),
            pltpu.VMEM((TPW, K + 1), jnp.int32),
            pltpu.SemaphoreType.DMA,
            pltpu.SemaphoreType.DMA,
        ],
        compiler_params=pltpu.CompilerParams(
            use_tc_tiling_on_sc=False, needs_layout_passes=False),
    )
    def _route(ids_hbm, tab_hbm, lg_hbm, w_hbm, i_hbm,
               idx_v, off_v, widx_v, win_v, lg_v, ow_v, oi_v, sem, sem2):
        wid = lax.axis_index("s") * _NC + lax.axis_index("c")
        base = wid * TPW
        lg_cp = pltpu.async_copy(
            lg_hbm.at[pl.ds(base * NE, TPW * NE)], lg_v, sem2)
        pltpu.sync_copy(ids_hbm.at[pl.ds(base, TPW)], idx_v)

        def wbody(g, c):
            t16 = jnp.arange(16, dtype=jnp.int32) + g * 16
            tok = idx_v[pl.ds(g * 16, 16)]
            w7 = tok * 7
            r = lax.shift_right_logical(w7, 3)
            off_v[pl.ds(g * 16, 16)] = jnp.bitwise_and(w7, 7)
            r2 = jnp.minimum(r + 1, TAB_ROWS - 1)
            p = t16 * 2
            plsc.store_scatter(
                widx_v,
                [lax.shift_right_logical(p, 7), jnp.bitwise_and(p, 127)], r)
            p1 = p + 1
            plsc.store_scatter(
                widx_v,
                [lax.shift_right_logical(p1, 7), jnp.bitwise_and(p1, 127)], r2)
            return c

        lax.fori_loop(0, TPW // 16, wbody, 0)

        cps = [pltpu.async_copy(tab_hbm.at[widx_v.at[j]], win_v.at[j], sem)
               for j in range(8)]
        for c in cps:
            c.wait()
        lg_cp.wait()

        def ebody(g, c):
            t16 = jnp.arange(16, dtype=jnp.int32) + g * 16
            fbase = t16 * 16 + off_v[pl.ds(g * 16, 16)]
            lbase = t16 * NE
            eids, ws = [], []
            for j in range(K):
                f = fbase + j
                eid = plsc.load_gather(
                    win_v,
                    [lax.shift_right_logical(f, 10),
                     jnp.bitwise_and(lax.shift_right_logical(f, 3), 127),
                     jnp.bitwise_and(f, 7)])
                x = plsc.load_gather(lg_v, [lbase + eid])
                eids.append(eid)
                ws.append(_sqrt_softplus(x))
            wsum = ws[0]
            for w in ws[1:]:
                wsum = wsum + w
            inv = 1.0 / wsum
            sn = jnp.zeros((16,), jnp.float32)
            for j in range(K):
                wn = ws[j] * inv
                sn = sn + wn
                cj = jnp.full((16,), j, jnp.int32)
                plsc.store_scatter(ow_v, [t16, cj], wn)
                plsc.store_scatter(oi_v, [t16, cj], eids[j])
            c7 = jnp.full((16,), K, jnp.int32)
            plsc.store_scatter(ow_v, [t16, c7], sn * INV_ROUTED_SCALING)
            plsc.store_scatter(oi_v, [t16, c7], jnp.full((16,), NE, jnp.int32))
            return c

        lax.fori_loop(0, TPW // 16, ebody, 0)

        pltpu.sync_copy(ow_v, w_hbm.at[pl.ds(base, TPW)])
        pltpu.sync_copy(oi_v, i_hbm.at[pl.ds(base, TPW)])

    return _route


def kernel(hidden_states, router_logits, input_ids, tid2eid):
    del hidden_states
    tab8 = tid2eid.reshape(TAB_ROWS, 8)
    lg_flat = router_logits.reshape(T * NE)
    topk_weights, topk_ids = _build_route()(input_ids, tab8, lg_flat)
    return topk_weights, topk_ids, router_logits
